# baseline (speedup 1.0000x reference)
import functools

import jax
import jax.numpy as jnp
from jax import lax
from jax.experimental import pallas as pl
from jax.experimental.pallas import tpu as pltpu
from jax.experimental.pallas import tpu_sc as plsc

EMB = 64
CHUNK = 128


@functools.lru_cache(maxsize=None)
def _build(n_rows: int):
    info = plsc.get_sparse_core_info()
    nw = info.num_cores * info.num_subcores
    per_w = n_rows // nw
    n_chunks = per_w // CHUNK
    assert per_w * nw == n_rows and n_chunks * CHUNK == per_w

    mesh = plsc.VectorSubcoreMesh(core_axis_name="c", subcore_axis_name="s")

    @functools.partial(
        pl.kernel,
        mesh=mesh,
        compiler_params=pltpu.CompilerParams(use_tc_tiling_on_sc=False),
        out_type=jax.ShapeDtypeStruct((n_rows, EMB), jnp.float32),
        scratch_types=[
            pltpu.VMEM((CHUNK,), jnp.int32),
            pltpu.VMEM((CHUNK,), jnp.int32),
            pltpu.VMEM((CHUNK,), jnp.int32),
            pltpu.VMEM((CHUNK, EMB), jnp.float32),
            pltpu.VMEM((CHUNK, EMB), jnp.float32),
            pltpu.VMEM((CHUNK, EMB), jnp.float32),
            pltpu.SemaphoreType.DMA,
        ],
    )
    def sc_kernel(qt, st, ct, qi, si, ci, out, qiv, siv, civ, qv, sv, cv, sem):
        wid = lax.axis_index("s") * info.num_cores + lax.axis_index("c")
        base = wid * per_w

        def chunk_body(g, carry):
            off = base + g * CHUNK
            pltpu.sync_copy(qi.at[pl.ds(off, CHUNK)], qiv)
            pltpu.sync_copy(si.at[pl.ds(off, CHUNK)], siv)
            pltpu.sync_copy(ci.at[pl.ds(off, CHUNK)], civ)
            dq = pltpu.async_copy(qt.at[qiv], qv, sem)
            ds_ = pltpu.async_copy(st.at[siv], sv, sem)
            dc = pltpu.async_copy(ct.at[civ], cv, sem)
            dq.wait()
            ds_.wait()
            dc.wait()

            def row_body(r, carry2):
                for cg in range(EMB // 16):
                    sl = pl.ds(cg * 16, 16)
                    qv[r, sl] = qv[r, sl] + sv[r, sl] + cv[r, sl]
                return carry2

            lax.fori_loop(0, CHUNK, row_body, 0, unroll=2)
            pltpu.sync_copy(qv, out.at[pl.ds(off, CHUNK)])
            return carry

        lax.fori_loop(0, n_chunks, chunk_body, 0)

    return sc_kernel


def kernel(qid_table, skill_table, correct_table, qid, skill, is_correct):
    b, l = qid.shape
    n = b * l
    fn = _build(n)
    out = fn(
        qid_table,
        skill_table,
        correct_table,
        qid.reshape(n).astype(jnp.int32),
        skill.reshape(n).astype(jnp.int32),
        is_correct.reshape(n).astype(jnp.int32),
    )
    return out.reshape(b, l, EMB)


# speedup vs baseline: 1.0004x; 1.0004x over previous
import functools

import jax
import jax.numpy as jnp
from jax import lax
from jax.experimental import pallas as pl
from jax.experimental.pallas import tpu as pltpu
from jax.experimental.pallas import tpu_sc as plsc

EMB = 64
SUB = 128
NBUF = 4


@functools.lru_cache(maxsize=None)
def _build(n_rows: int):
    info = plsc.get_sparse_core_info()
    nw = info.num_cores * info.num_subcores
    per_w = n_rows // nw
    blk_rows = NBUF * SUB
    n_blk = per_w // blk_rows
    assert per_w * nw == n_rows and n_blk * blk_rows == per_w

    mesh = plsc.VectorSubcoreMesh(core_axis_name="c", subcore_axis_name="s")

    @functools.partial(
        pl.kernel,
        mesh=mesh,
        compiler_params=pltpu.CompilerParams(use_tc_tiling_on_sc=False),
        out_type=jax.ShapeDtypeStruct((n_rows, EMB), jnp.float32),
        scratch_types=[
            pltpu.VMEM((2, blk_rows), jnp.int32),
            pltpu.VMEM((2, blk_rows), jnp.int32),
            pltpu.VMEM((2, blk_rows), jnp.int32),
            pltpu.VMEM((NBUF, SUB, EMB), jnp.float32),
            pltpu.VMEM((NBUF, SUB, EMB), jnp.float32),
            pltpu.VMEM((NBUF, SUB, EMB), jnp.float32),
            [pltpu.SemaphoreType.DMA] * NBUF,
            [pltpu.SemaphoreType.DMA] * NBUF,
            pltpu.SemaphoreType.DMA,
        ],
    )
    def sc_kernel(qt, st, ct, qi, si, ci, out, qiv, siv, civ, qb, sb, cb,
                  gsem, wsem, isem):
        wid = lax.axis_index("s") * info.num_cores + lax.axis_index("c")
        base = wid * per_w

        def fire_idx(blk, par):
            off = base + blk * blk_rows
            pltpu.async_copy(qi.at[pl.ds(off, blk_rows)], qiv.at[par], isem)
            pltpu.async_copy(si.at[pl.ds(off, blk_rows)], siv.at[par], isem)
            pltpu.async_copy(ci.at[pl.ds(off, blk_rows)], civ.at[par], isem)

        def wait_idx():
            for ref in (qiv, siv, civ):
                pltpu.make_async_copy(qi.at[pl.ds(0, blk_rows)],
                                      ref.at[0], isem).wait()

        def fire_gathers(b, par):
            sl = pl.ds(b * SUB, SUB)
            pltpu.async_copy(qt.at[qiv.at[par, sl]], qb.at[b], gsem[b])
            pltpu.async_copy(st.at[siv.at[par, sl]], sb.at[b], gsem[b])
            pltpu.async_copy(ct.at[civ.at[par, sl]], cb.at[b], gsem[b])

        def wait_gathers(b):
            for buf in (qb, sb, cb):
                pltpu.make_async_copy(qt.at[qiv.at[0, pl.ds(0, SUB)]],
                                      buf.at[b], gsem[b]).wait()

        fire_idx(0, 0)
        wait_idx()
        if n_blk > 1:
            fire_idx(1, 1)
        for b in range(NBUF):
            fire_gathers(b, 0)

        def blk_body(blk, carry):
            par = lax.rem(blk, 2)
            for b in range(NBUF):
                wait_gathers(b)

                def row_body(r, c2):
                    for cg in range(EMB // 16):
                        s2 = pl.ds(cg * 16, 16)
                        qb[b, r, s2] = qb[b, r, s2] + sb[b, r, s2] + cb[b, r, s2]
                    return c2

                lax.fori_loop(0, SUB, row_body, 0, unroll=4)
                off = base + blk * blk_rows + b * SUB
                pltpu.async_copy(qb.at[b], out.at[pl.ds(off, SUB)], wsem[b])

            @pl.when(blk < n_blk - 1)
            def _():
                wait_idx()

                @pl.when(blk < n_blk - 2)
                def _():
                    fire_idx(blk + 2, par)

                for b in range(NBUF):
                    pltpu.make_async_copy(
                        qb.at[b], out.at[pl.ds(0, SUB)], wsem[b]).wait()
                    fire_gathers(b, 1 - par)
            return carry

        lax.fori_loop(0, n_blk, blk_body, 0)
        for b in range(NBUF):
            pltpu.make_async_copy(qb.at[b], out.at[pl.ds(0, SUB)],
                                  wsem[b]).wait()

    return sc_kernel


def kernel(qid_table, skill_table, correct_table, qid, skill, is_correct):
    b, l = qid.shape
    n = b * l
    fn = _build(n)
    out = fn(
        qid_table,
        skill_table,
        correct_table,
        qid.reshape(n).astype(jnp.int32),
        skill.reshape(n).astype(jnp.int32),
        is_correct.reshape(n).astype(jnp.int32),
    )
    return out.reshape(b, l, EMB)


# speedup vs baseline: 1.8726x; 1.8718x over previous
import functools

import jax
import jax.numpy as jnp
from jax import lax
from jax.experimental import pallas as pl
from jax.experimental.pallas import tpu as pltpu
from jax.experimental.pallas import tpu_sc as plsc

EMB = 64
SUB = 128
NBUF = 4
SKILL_PAD = 1024
N_CORR = 4


@functools.lru_cache(maxsize=None)
def _build(n_rows: int):
    info = plsc.get_sparse_core_info()
    nw = info.num_cores * info.num_subcores
    per_w = n_rows // nw
    blk_rows = NBUF * SUB
    n_blk = per_w // blk_rows
    assert per_w * nw == n_rows and n_blk * blk_rows == per_w

    mesh = plsc.VectorSubcoreMesh(core_axis_name="c", subcore_axis_name="s")

    @functools.partial(
        pl.kernel,
        mesh=mesh,
        compiler_params=pltpu.CompilerParams(use_tc_tiling_on_sc=False, needs_layout_passes=False),
        out_type=jax.ShapeDtypeStruct((n_rows, EMB), jnp.float32),
        scratch_types=[
            pltpu.VMEM((2, blk_rows), jnp.int32),
            pltpu.VMEM((2, blk_rows), jnp.int32),
            pltpu.VMEM((NBUF, SUB, EMB), jnp.float32),
            pltpu.VMEM((SKILL_PAD, EMB), jnp.float32),
            pltpu.VMEM((N_CORR, EMB), jnp.float32),
            [pltpu.SemaphoreType.DMA] * NBUF,
            [pltpu.SemaphoreType.DMA] * NBUF,
            pltpu.SemaphoreType.DMA,
        ],
    )
    def sc_kernel(qt, st, ct, qi, sci, out, qiv, sciv, qb, skv, cov,
                  gsem, wsem, isem):
        wid = lax.axis_index("s") * info.num_cores + lax.axis_index("c")
        base = wid * per_w

        pltpu.sync_copy(st, skv)
        pltpu.sync_copy(ct, cov)

        def fire_idx(blk, par):
            off = base + blk * blk_rows
            pltpu.async_copy(qi.at[pl.ds(off, blk_rows)], qiv.at[par], isem)
            pltpu.async_copy(sci.at[pl.ds(off, blk_rows)], sciv.at[par], isem)

        def wait_idx():
            for ref in (qiv, sciv):
                pltpu.make_async_copy(qi.at[pl.ds(0, blk_rows)],
                                      ref.at[0], isem).wait()

        def fire_gathers(b, par):
            sl = pl.ds(b * SUB, SUB)
            pltpu.async_copy(qt.at[qiv.at[par, sl]], qb.at[b], gsem[b])

        def wait_gathers(b):
            pltpu.make_async_copy(qt.at[qiv.at[0, pl.ds(0, SUB)]],
                                  qb.at[b], gsem[b]).wait()

        fire_idx(0, 0)
        wait_idx()
        if n_blk > 1:
            fire_idx(1, 1)
        for b in range(NBUF):
            fire_gathers(b, 0)

        lanes = lax.iota(jnp.int32, 16)

        def blk_body(blk, carry):
            par = lax.rem(blk, 2)
            for b in range(NBUF):
                wait_gathers(b)
                bb = jnp.full((16,), b, jnp.int32)

                def grp_body(gi, c1):
                    r0 = pl.multiple_of(gi * 16, 16)
                    scv = sciv[par, pl.ds(b * SUB + r0, 16)]
                    s_idx = lax.shift_right_logical(scv, 2)
                    c_idx = lax.bitwise_and(scv, 3)
                    rows = r0 + lanes

                    def col_body(c, c2):
                        cc = jnp.full((16,), 0, jnp.int32) + c
                        q = plsc.load_gather(qb, [bb, rows, cc])
                        sk = plsc.load_gather(skv, [s_idx, cc])
                        co = plsc.load_gather(cov, [c_idx, cc])
                        plsc.store_scatter(qb, [bb, rows, cc], q + sk + co)
                        return c2

                    lax.fori_loop(0, EMB, col_body, 0, unroll=8)
                    return c1

                lax.fori_loop(0, SUB // 16, grp_body, 0)
                off = base + blk * blk_rows + b * SUB
                pltpu.async_copy(qb.at[b], out.at[pl.ds(off, SUB)], wsem[b])

            @pl.when(blk < n_blk - 1)
            def _():
                wait_idx()

                @pl.when(blk < n_blk - 2)
                def _():
                    fire_idx(blk + 2, par)

                for b in range(NBUF):
                    pltpu.make_async_copy(
                        qb.at[b], out.at[pl.ds(0, SUB)], wsem[b]).wait()
                    fire_gathers(b, 1 - par)
            return carry

        lax.fori_loop(0, n_blk, blk_body, 0)
        for b in range(NBUF):
            pltpu.make_async_copy(qb.at[b], out.at[pl.ds(0, SUB)],
                                  wsem[b]).wait()

    return sc_kernel


def kernel(qid_table, skill_table, correct_table, qid, skill, is_correct):
    b, l = qid.shape
    n = b * l
    fn = _build(n)
    skill_p = (jnp.zeros((SKILL_PAD, EMB), jnp.float32)
               .at[:skill_table.shape[0]].set(skill_table))
    sc_idx = skill.astype(jnp.int32) * N_CORR + is_correct.astype(jnp.int32)
    out = fn(
        qid_table,
        skill_p,
        correct_table,
        qid.reshape(n).astype(jnp.int32),
        sc_idx.reshape(n),
    )
    return out.reshape(b, l, EMB)


# speedup vs baseline: 2.1701x; 1.1588x over previous
import functools

import jax
import jax.numpy as jnp
from jax import lax
from jax.experimental import pallas as pl
from jax.experimental.pallas import tpu as pltpu
from jax.experimental.pallas import tpu_sc as plsc

EMB = 64
SUB = 128
NBUF = 4
SKILL_PAD = 1024
N_CORR = 4


@functools.lru_cache(maxsize=None)
def _build(n_rows: int):
    info = plsc.get_sparse_core_info()
    nw = info.num_cores * info.num_subcores
    per_w = n_rows // nw
    blk_rows = NBUF * SUB
    n_blk = per_w // blk_rows
    assert per_w * nw == n_rows and n_blk * blk_rows == per_w

    mesh = plsc.VectorSubcoreMesh(core_axis_name="c", subcore_axis_name="s")

    @functools.partial(
        pl.kernel,
        mesh=mesh,
        compiler_params=pltpu.CompilerParams(use_tc_tiling_on_sc=False, needs_layout_passes=False),
        out_type=jax.ShapeDtypeStruct((n_rows, EMB), jnp.float32),
        scratch_types=[
            pltpu.VMEM((2, blk_rows), jnp.int32),
            pltpu.VMEM((2, blk_rows), jnp.int32),
            pltpu.VMEM((NBUF, SUB, EMB), jnp.float32),
            pltpu.VMEM((SKILL_PAD, EMB), jnp.float32),
            pltpu.VMEM((N_CORR, EMB), jnp.float32),
            [pltpu.SemaphoreType.DMA] * NBUF,
            [pltpu.SemaphoreType.DMA] * NBUF,
            pltpu.SemaphoreType.DMA,
        ],
    )
    def sc_kernel(qt, st, ct, qi, sci, out, qiv, sciv, qb, skv, cov,
                  gsem, wsem, isem):
        wid = lax.axis_index("s") * info.num_cores + lax.axis_index("c")
        base = wid * per_w

        pltpu.sync_copy(st, skv)
        pltpu.sync_copy(ct, cov)

        def fire_idx(blk, par):
            off = base + blk * blk_rows
            pltpu.async_copy(qi.at[pl.ds(off, blk_rows)], qiv.at[par], isem)
            pltpu.async_copy(sci.at[pl.ds(off, blk_rows)], sciv.at[par], isem)

        def wait_idx():
            for ref in (qiv, sciv):
                pltpu.make_async_copy(qi.at[pl.ds(0, blk_rows)],
                                      ref.at[0], isem).wait()

        def fire_gathers(b, par):
            sl = pl.ds(b * SUB, SUB)
            pltpu.async_copy(qt.at[qiv.at[par, sl]], qb.at[b], gsem[b])

        def wait_gathers(b):
            pltpu.make_async_copy(qt.at[qiv.at[0, pl.ds(0, SUB)]],
                                  qb.at[b], gsem[b]).wait()

        fire_idx(0, 0)
        wait_idx()
        if n_blk > 1:
            fire_idx(1, 1)
        for b in range(NBUF):
            fire_gathers(b, 0)

        lanes = lax.iota(jnp.int32, 16)

        def blk_body(blk, carry):
            par = lax.rem(blk, 2)
            for b in range(NBUF):
                wait_gathers(b)
                bb = jnp.full((16,), b, jnp.int32)

                def grp_body(gi, c1):
                    r0 = pl.multiple_of(gi * 16, 16)
                    scv = sciv[par, pl.ds(b * SUB + r0, 16)]
                    s_idx = lax.shift_right_logical(scv, 2)
                    c_idx = lax.bitwise_and(scv, 3)
                    rows = r0 + lanes

                    def col_body(c, c2):
                        cc = jnp.full((16,), 0, jnp.int32) + c
                        sk = plsc.load_gather(skv, [s_idx, cc])
                        co = plsc.load_gather(cov, [c_idx, cc])
                        plsc.addupdate_scatter(qb, [bb, rows, cc], sk + co)
                        return c2

                    lax.fori_loop(0, EMB, col_body, 0, unroll=8)
                    return c1

                lax.fori_loop(0, SUB // 16, grp_body, 0)
                off = base + blk * blk_rows + b * SUB
                pltpu.async_copy(qb.at[b], out.at[pl.ds(off, SUB)], wsem[b])

            @pl.when(blk < n_blk - 1)
            def _():
                wait_idx()

                @pl.when(blk < n_blk - 2)
                def _():
                    fire_idx(blk + 2, par)

                for b in range(NBUF):
                    pltpu.make_async_copy(
                        qb.at[b], out.at[pl.ds(0, SUB)], wsem[b]).wait()
                    fire_gathers(b, 1 - par)
            return carry

        lax.fori_loop(0, n_blk, blk_body, 0)
        for b in range(NBUF):
            pltpu.make_async_copy(qb.at[b], out.at[pl.ds(0, SUB)],
                                  wsem[b]).wait()

    return sc_kernel


def kernel(qid_table, skill_table, correct_table, qid, skill, is_correct):
    b, l = qid.shape
    n = b * l
    fn = _build(n)
    skill_p = (jnp.zeros((SKILL_PAD, EMB), jnp.float32)
               .at[:skill_table.shape[0]].set(skill_table))
    sc_idx = skill.astype(jnp.int32) * N_CORR + is_correct.astype(jnp.int32)
    out = fn(
        qid_table,
        skill_p,
        correct_table,
        qid.reshape(n).astype(jnp.int32),
        sc_idx.reshape(n),
    )
    return out.reshape(b, l, EMB)


# speedup vs baseline: 2.7007x; 1.2445x over previous
import functools

import jax
import jax.numpy as jnp
from jax import lax
from jax.experimental import pallas as pl
from jax.experimental.pallas import tpu as pltpu
from jax.experimental.pallas import tpu_sc as plsc

EMB = 64
SUB = 128
NBUF = 4
SKILL_PAD = 1024
N_CORR = 4


@functools.lru_cache(maxsize=None)
def _build(n_rows: int):
    info = plsc.get_sparse_core_info()
    nw = info.num_cores * info.num_subcores
    per_w = n_rows // nw
    blk_rows = NBUF * SUB
    n_blk = per_w // blk_rows
    assert per_w * nw == n_rows and n_blk * blk_rows == per_w

    mesh = plsc.VectorSubcoreMesh(core_axis_name="c", subcore_axis_name="s")

    @functools.partial(
        pl.kernel,
        mesh=mesh,
        compiler_params=pltpu.CompilerParams(use_tc_tiling_on_sc=False, needs_layout_passes=False),
        out_type=jax.ShapeDtypeStruct((n_rows, EMB), jnp.float32),
        scratch_types=[
            pltpu.VMEM((2, blk_rows), jnp.int32),
            pltpu.VMEM((2, blk_rows), jnp.int32),
            pltpu.VMEM((NBUF, SUB, EMB), jnp.float32),
            pltpu.VMEM((SKILL_PAD, EMB), jnp.float32),
            pltpu.VMEM((N_CORR, EMB), jnp.float32),
            [pltpu.SemaphoreType.DMA] * NBUF,
            [pltpu.SemaphoreType.DMA] * NBUF,
            pltpu.SemaphoreType.DMA,
        ],
    )
    def sc_kernel(qt, st, ct, qi, sci, out, qiv, sciv, qb, skv, cov,
                  gsem, wsem, isem):
        wid = lax.axis_index("s") * info.num_cores + lax.axis_index("c")
        base = wid * per_w

        pltpu.sync_copy(st, skv)
        pltpu.sync_copy(ct, cov)

        def fire_idx(blk, par):
            off = base + blk * blk_rows
            pltpu.async_copy(qi.at[pl.ds(off, blk_rows)], qiv.at[par], isem)
            pltpu.async_copy(sci.at[pl.ds(off, blk_rows)], sciv.at[par], isem)

        def wait_idx():
            for ref in (qiv, sciv):
                pltpu.make_async_copy(qi.at[pl.ds(0, blk_rows)],
                                      ref.at[0], isem).wait()

        def fire_gathers(b, par):
            sl = pl.ds(b * SUB, SUB)
            pltpu.async_copy(qt.at[qiv.at[par, sl]], qb.at[b], gsem[b])

        def wait_gathers(b):
            pltpu.make_async_copy(qt.at[qiv.at[0, pl.ds(0, SUB)]],
                                  qb.at[b], gsem[b]).wait()

        fire_idx(0, 0)
        wait_idx()
        if n_blk > 1:
            fire_idx(1, 1)
        for b in range(NBUF):
            fire_gathers(b, 0)

        lanes = lax.iota(jnp.int32, 16)

        def blk_body(blk, carry):
            par = lax.rem(blk, 2)
            for b in range(NBUF):
                wait_gathers(b)
                bb = jnp.full((16,), b, jnp.int32)

                def grp_body(gi, c1):
                    r0 = pl.multiple_of(gi * 16, 16)
                    scv = sciv[par, pl.ds(b * SUB + r0, 16)]
                    s_idx = lax.shift_right_logical(scv, 2)
                    c_idx = lax.bitwise_and(scv, 3)
                    rows = r0 + lanes

                    @plsc.parallel_loop(0, EMB, unroll=8)
                    def col_body(c):
                        cc = jnp.full((16,), 0, jnp.int32) + c
                        sk = plsc.load_gather(skv, [s_idx, cc])
                        co = plsc.load_gather(cov, [c_idx, cc])
                        plsc.addupdate_scatter(qb, [bb, rows, cc], sk + co)

                    return c1

                lax.fori_loop(0, SUB // 16, grp_body, 0)
                off = base + blk * blk_rows + b * SUB
                pltpu.async_copy(qb.at[b], out.at[pl.ds(off, SUB)], wsem[b])

            @pl.when(blk < n_blk - 1)
            def _():
                wait_idx()

                @pl.when(blk < n_blk - 2)
                def _():
                    fire_idx(blk + 2, par)

                for b in range(NBUF):
                    pltpu.make_async_copy(
                        qb.at[b], out.at[pl.ds(0, SUB)], wsem[b]).wait()
                    fire_gathers(b, 1 - par)
            return carry

        lax.fori_loop(0, n_blk, blk_body, 0)
        for b in range(NBUF):
            pltpu.make_async_copy(qb.at[b], out.at[pl.ds(0, SUB)],
                                  wsem[b]).wait()

    return sc_kernel


def kernel(qid_table, skill_table, correct_table, qid, skill, is_correct):
    b, l = qid.shape
    n = b * l
    fn = _build(n)
    skill_p = (jnp.zeros((SKILL_PAD, EMB), jnp.float32)
               .at[:skill_table.shape[0]].set(skill_table))
    sc_idx = skill.astype(jnp.int32) * N_CORR + is_correct.astype(jnp.int32)
    out = fn(
        qid_table,
        skill_p,
        correct_table,
        qid.reshape(n).astype(jnp.int32),
        sc_idx.reshape(n),
    )
    return out.reshape(b, l, EMB)


# speedup vs baseline: 6.5384x; 2.4210x over previous
import functools

import jax
import jax.numpy as jnp
from jax import lax
from jax.experimental import pallas as pl
from jax.experimental.pallas import tpu as pltpu
from jax.experimental.pallas import tpu_sc as plsc

EMB = 64
SUB = 128
NBUF = 4
SKILL_PAD = 1024
N_CORR = 4


@functools.lru_cache(maxsize=None)
def _build(n_rows: int):
    info = plsc.get_sparse_core_info()
    nw = info.num_cores * info.num_subcores
    per_w = n_rows // nw
    blk_rows = NBUF * SUB
    n_blk = per_w // blk_rows
    assert per_w * nw == n_rows and n_blk * blk_rows == per_w

    mesh = plsc.VectorSubcoreMesh(core_axis_name="c", subcore_axis_name="s")

    @functools.partial(
        pl.kernel,
        mesh=mesh,
        compiler_params=pltpu.CompilerParams(use_tc_tiling_on_sc=False, needs_layout_passes=False),
        out_type=jax.ShapeDtypeStruct((n_rows, EMB), jnp.float32),
        scratch_types=[
            pltpu.VMEM((2, blk_rows), jnp.int32),
            pltpu.VMEM((2, blk_rows), jnp.int32),
            pltpu.VMEM((NBUF, SUB, EMB), jnp.float32),
            pltpu.VMEM((EMB, SKILL_PAD + 1), jnp.float32),
            pltpu.VMEM((EMB, N_CORR + 1), jnp.float32),
            [pltpu.SemaphoreType.DMA] * NBUF,
            [pltpu.SemaphoreType.DMA] * NBUF,
            pltpu.SemaphoreType.DMA,
        ],
    )
    def sc_kernel(qt, st, ct, qi, sci, out, qiv, sciv, qb, skv, cov,
                  gsem, wsem, isem):
        wid = lax.axis_index("s") * info.num_cores + lax.axis_index("c")
        base = wid * per_w

        pltpu.sync_copy(st, skv)
        pltpu.sync_copy(ct, cov)

        def fire_idx(blk, par):
            off = base + blk * blk_rows
            pltpu.async_copy(qi.at[pl.ds(off, blk_rows)], qiv.at[par], isem)
            pltpu.async_copy(sci.at[pl.ds(off, blk_rows)], sciv.at[par], isem)

        def wait_idx():
            for ref in (qiv, sciv):
                pltpu.make_async_copy(qi.at[pl.ds(0, blk_rows)],
                                      ref.at[0], isem).wait()

        def fire_gathers(b, par):
            sl = pl.ds(b * SUB, SUB)
            pltpu.async_copy(qt.at[qiv.at[par, sl]], qb.at[b], gsem[b])

        def wait_gathers(b):
            pltpu.make_async_copy(qt.at[qiv.at[0, pl.ds(0, SUB)]],
                                  qb.at[b], gsem[b]).wait()

        fire_idx(0, 0)
        wait_idx()
        if n_blk > 1:
            fire_idx(1, 1)
        for b in range(NBUF):
            fire_gathers(b, 0)

        lanes = lax.iota(jnp.int32, 16)

        def blk_body(blk, carry):
            par = lax.rem(blk, 2)
            for b in range(NBUF):
                wait_gathers(b)
                bb = jnp.full((16,), b, jnp.int32)

                def grp_body(gi, c1):
                    r0 = pl.multiple_of(gi * 16, 16)
                    scv = sciv[par, pl.ds(b * SUB + r0, 16)]
                    s_idx = lax.shift_right_logical(scv, 2)
                    c_idx = lax.bitwise_and(scv, 3)
                    rows = r0 + lanes

                    @plsc.parallel_loop(0, EMB, unroll=8)
                    def col_body(c):
                        cc = lax.bitwise_and(lanes + c, EMB - 1)
                        sk = plsc.load_gather(skv, [cc, s_idx])
                        co = plsc.load_gather(cov, [cc, c_idx])
                        plsc.addupdate_scatter(qb, [bb, rows, cc], sk + co)

                    return c1

                lax.fori_loop(0, SUB // 16, grp_body, 0)
                off = base + blk * blk_rows + b * SUB
                pltpu.async_copy(qb.at[b], out.at[pl.ds(off, SUB)], wsem[b])

            @pl.when(blk < n_blk - 1)
            def _():
                wait_idx()

                @pl.when(blk < n_blk - 2)
                def _():
                    fire_idx(blk + 2, par)

                for b in range(NBUF):
                    pltpu.make_async_copy(
                        qb.at[b], out.at[pl.ds(0, SUB)], wsem[b]).wait()
                    fire_gathers(b, 1 - par)
            return carry

        lax.fori_loop(0, n_blk, blk_body, 0)
        for b in range(NBUF):
            pltpu.make_async_copy(qb.at[b], out.at[pl.ds(0, SUB)],
                                  wsem[b]).wait()

    return sc_kernel


def kernel(qid_table, skill_table, correct_table, qid, skill, is_correct):
    b, l = qid.shape
    n = b * l
    fn = _build(n)
    skill_p = (jnp.zeros((SKILL_PAD, EMB), jnp.float32)
               .at[:skill_table.shape[0]].set(skill_table))
    sc_idx = skill.astype(jnp.int32) * N_CORR + is_correct.astype(jnp.int32)
    skill_t = (jnp.zeros((EMB, SKILL_PAD + 1), jnp.float32)
               .at[:, :SKILL_PAD].set(skill_p.T))
    correct_t = (jnp.zeros((EMB, N_CORR + 1), jnp.float32)
                 .at[:, :N_CORR].set(correct_table.T))
    out = fn(
        qid_table,
        skill_t,
        correct_t,
        qid.reshape(n).astype(jnp.int32),
        sc_idx.reshape(n),
    )
    return out.reshape(b, l, EMB)
